# baseline (speedup 1.0000x reference)
import jax
import jax.numpy as jnp
from jax import lax
from jax.experimental import pallas as pl
from jax.experimental.pallas import tpu as pltpu
from jax.experimental.pallas import tpu_sc as plsc

H = 8
DH = 64
TOPK = 8
EMB = 512
M = 8192
D = H * DH
NEG = -3.0e38

CHUNK = 128
NCHUNK = M // CHUNK
NWORK = 32
L = 16


def _proj_body(x_ref, wq_ref, wk_ref, wv_ref, bq_ref, bk_ref, bv_ref,
               qf_ref, ckv_ref):
    xb = x_ref[...]
    q = jnp.dot(xb, wq_ref[...], preferred_element_type=jnp.float32) + bq_ref[...]
    k = jnp.dot(xb, wk_ref[...], preferred_element_type=jnp.float32) + bk_ref[...]
    v = jnp.dot(xb, wv_ref[...], preferred_element_type=jnp.float32) + bv_ref[...]
    qn = jnp.sqrt(jnp.sum(q * q, axis=1, keepdims=True))
    kn = jnp.sqrt(jnp.sum(k * k, axis=1, keepdims=True))
    qf_ref[...] = q / jnp.maximum(qn, 1e-12)
    ckv_ref[0, :, 0, :] = k / jnp.maximum(kn, 1e-12)
    ckv_ref[0, :, 1, :] = v


def _projections(xf, WqT, WkT, WvT, bq, bk, bv, b, s):
    n = xf.shape[0]
    blk = 256
    per_b = s // blk
    row_spec = pl.BlockSpec((blk, EMB), lambda i: (i, 0))
    w_spec = pl.BlockSpec((EMB, D), lambda i: (0, 0))
    b_spec = pl.BlockSpec((1, D), lambda i: (0, 0))
    return pl.pallas_call(
        _proj_body,
        grid=(n // blk,),
        in_specs=[row_spec, w_spec, w_spec, w_spec, b_spec, b_spec, b_spec],
        out_specs=[pl.BlockSpec((blk, D), lambda i: (i, 0)),
                   pl.BlockSpec((1, blk, 2, D),
                                lambda i: (i // per_b, i % per_b, 0, 0))],
        out_shape=[jax.ShapeDtypeStruct((n, D), jnp.float32),
                   jax.ShapeDtypeStruct((b, s, 2, D), jnp.float32)],
    )(xf, WqT, WkT, WvT, bq, bk, bv)


def _scores_body(qf_ref, keys_ref, s_ref, cm_ref):
    s = jax.lax.dot_general(qf_ref[...], keys_ref[...],
                            (((1,), (1,)), ((), ())),
                            preferred_element_type=jnp.float32)
    blk = s.shape[0]
    s_ref[...] = s.reshape(blk * NCHUNK, CHUNK)
    cm_ref[...] = jnp.max(s.reshape(blk, NCHUNK, CHUNK), axis=2).reshape(
        blk * NCHUNK // CHUNK, CHUNK)


def _knn_scores(qf, keys):
    n = qf.shape[0]
    blk = 64
    return pl.pallas_call(
        _scores_body,
        grid=(n // blk,),
        in_specs=[pl.BlockSpec((blk, D), lambda i: (i, 0)),
                  pl.BlockSpec((M, D), lambda i: (0, 0))],
        out_specs=[pl.BlockSpec((blk * NCHUNK, CHUNK), lambda i: (i, 0)),
                   pl.BlockSpec((blk * NCHUNK // CHUNK, CHUNK),
                                lambda i: (i, 0))],
        out_shape=[jax.ShapeDtypeStruct((n * NCHUNK, CHUNK), jnp.float32),
                   jax.ShapeDtypeStruct((n * NCHUNK // CHUNK, CHUNK),
                                        jnp.float32)],
    )(qf, keys)


def _sc_topk_gather_body(scores2_hbm, cm2_hbm, kv_hbm, kx_hbm, vx_hbm,
                         cmb, cbuf, cidx, cms, kvidx, gA, gB,
                         sgA, sgB, soA, soB):
    rpw = 2048 // NWORK
    npair = rpw // 2
    wid = lax.axis_index("s") * 2 + lax.axis_index("c")
    base = wid * rpw
    iota = lax.iota(jnp.int32, L)
    big = jnp.int32(1 << 30)
    cvec = NCHUNK // L

    pltpu.sync_copy(cm2_hbm.at[pl.ds(wid * npair, npair)], cmb)

    def sel_pair(p, carry):
        accP = jnp.zeros((L,), jnp.int32)
        cmsP = jnp.full((L,), NEG, jnp.float32)
        for half in range(2):
            r = 2 * p + half
            lb = TOPK * half
            cw = [cmb[p, pl.ds(half * NCHUNK + L * j, L)] for j in range(cvec)]
            for t in range(TOPK):
                m = jnp.max(jnp.maximum(jnp.maximum(cw[0], cw[1]),
                                        jnp.maximum(cw[2], cw[3])))
                cand = jnp.where(cw[0] == m, iota, big)
                for j in range(1, cvec):
                    cand = jnp.minimum(cand, jnp.where(cw[j] == m, iota + L * j, big))
                c = jnp.min(cand)
                accP = jnp.where(iota == lb + t, (base + r) * NCHUNK + c, accP)
                cmsP = jnp.where(iota == lb + t, m, cmsP)
                for j in range(cvec):
                    cw[j] = jnp.where(iota == c - L * j, NEG, cw[j])
        cidx[pl.ds(p * L, L)] = accP
        cms[pl.ds(p * L, L)] = cmsP
        return carry

    lax.fori_loop(0, npair, sel_pair, 0)

    for hf in range(2):
        po = hf * (npair // 2)
        coff = po * L
        nf = npair // 2 * L
        for qd in range(nf // CHUNK):
            pltpu.async_copy(
                scores2_hbm.at[cidx.at[pl.ds(coff + qd * CHUNK, CHUNK)]],
                cbuf.at[pl.ds(qd * CHUNK, CHUNK)], sgA)
        for qd in range(nf // CHUNK):
            pltpu.make_async_copy(
                scores2_hbm.at[cidx.at[pl.ds(coff + qd * CHUNK, CHUNK)]],
                cbuf.at[pl.ds(qd * CHUNK, CHUNK)], sgA).wait()

        def top_pair(p, carry, po=po):
            cmsv = cms[pl.ds(p * L, L)]
            fidv = cidx[pl.ds(p * L, L)]
            accK = jnp.zeros((L,), jnp.int32)
            for half in range(2):
                r = 2 * p + half
                lb = TOPK * half
                inr = (iota >= lb) & (iota < lb + TOPK)
                cm8 = jnp.where(inr, cmsv, NEG)
                for t in range(TOPK):
                    m = jnp.max(cm8)
                    c = jnp.min(jnp.where(cm8 == m, iota, big))
                    slot = (p - po) * L + c
                    fid = jnp.min(jnp.where(iota == c, fidv, big))
                    c_loc = fid - (base + r) * NCHUNK
                    vs = [cbuf[slot, pl.ds(L * j, L)] for j in range(CHUNK // L)]
                    ecand = jnp.where(vs[0] == m, iota, big)
                    for j in range(1, CHUNK // L):
                        ecand = jnp.minimum(ecand,
                                            jnp.where(vs[j] == m, iota + L * j, big))
                    e = jnp.min(ecand)
                    accK = jnp.where(iota == lb + t, c_loc * CHUNK + e, accK)
                    nmv = None
                    for j in range(CHUNK // L):
                        vj = jnp.where(iota == e - L * j, NEG, vs[j])
                        cbuf[slot, pl.ds(L * j, L)] = vj
                        nmv = vj if nmv is None else jnp.maximum(nmv, vj)
                    cm8 = jnp.where(iota == c, jnp.max(nmv), cm8)
            kvidx[pl.ds(p * L, L)] = accK
            return carry

        lax.fori_loop(po, po + npair // 2, top_pair, 0)

    def issue_gather(p, buf, sem):
        pltpu.async_copy(kv_hbm.at[kvidx.at[pl.ds(p * L, L)]], buf, sem)

    def drain_gather(p, buf, sem):
        pltpu.make_async_copy(kv_hbm.at[kvidx.at[pl.ds(p * L, L)]], buf, sem).wait()

    def issue_out(p, buf, sem):
        dst = pl.ds((base + 2 * p) * TOPK, 2 * TOPK)
        pltpu.async_copy(buf.at[:, 0, :], kx_hbm.at[dst], sem)
        pltpu.async_copy(buf.at[:, 1, :], vx_hbm.at[dst], sem)

    def drain_out(p, buf, sem):
        dst = pl.ds((base + 2 * p) * TOPK, 2 * TOPK)
        pltpu.make_async_copy(buf.at[:, 0, :], kx_hbm.at[dst], sem).wait()
        pltpu.make_async_copy(buf.at[:, 1, :], vx_hbm.at[dst], sem).wait()

    issue_gather(0, gA, sgA)

    def g_body(i, carry):
        pA = 2 * i
        pB = 2 * i + 1
        issue_gather(pB, gB, sgB)
        drain_gather(pA, gA, sgA)
        issue_out(pA, gA, soA)
        drain_gather(pB, gB, sgB)
        issue_out(pB, gB, soB)
        drain_out(pA, gA, soA)

        @pl.when(i < npair // 2 - 1)
        def _():
            issue_gather(pA + 2, gA, sgA)
        drain_out(pB, gB, soB)
        return carry

    lax.fori_loop(0, npair // 2, g_body, 0)


def _sc_topk_gather(scores2, cm2, kv_mem):
    n = scores2.shape[0] // NCHUNK
    rpw = n // NWORK
    mesh = plsc.VectorSubcoreMesh(core_axis_name="c", subcore_axis_name="s")
    fn = pl.kernel(
        _sc_topk_gather_body,
        out_type=[jax.ShapeDtypeStruct((n * TOPK, D), jnp.float32),
                  jax.ShapeDtypeStruct((n * TOPK, D), jnp.float32)],
        mesh=mesh,
        scratch_types=[
            pltpu.VMEM((rpw // 2, CHUNK), jnp.float32),
            pltpu.VMEM((rpw * TOPK // 2, CHUNK), jnp.float32),
            pltpu.VMEM((rpw * TOPK,), jnp.int32),
            pltpu.VMEM((rpw * TOPK,), jnp.float32),
            pltpu.VMEM((rpw * TOPK,), jnp.int32),
            pltpu.VMEM((2 * TOPK, 2, D), jnp.float32),
            pltpu.VMEM((2 * TOPK, 2, D), jnp.float32),
            pltpu.SemaphoreType.DMA,
            pltpu.SemaphoreType.DMA,
            pltpu.SemaphoreType.DMA,
            pltpu.SemaphoreType.DMA,
        ],
        compiler_params=pltpu.CompilerParams(needs_layout_passes=False),
    )
    return fn(scores2, cm2, kv_mem)


def _attn_body(gate_ref, q_ref, ckv_ref, kx_ref, vx_ref, o_ref):
    scale = DH ** -0.5
    q = q_ref[0] * scale
    k = ckv_ref[0, :, 0, :]
    v = ckv_ref[0, :, 1, :]
    kx = kx_ref[0]
    vx = vx_ref[0]
    s = q.shape[0]
    ri = jax.lax.broadcasted_iota(jnp.int32, (s, s), 0)
    ci = jax.lax.broadcasted_iota(jnp.int32, (s, s), 1)
    causal = ci <= ri
    outs = []
    for h in range(H):
        sl = slice(h * DH, (h + 1) * DH)
        qh = q[:, sl]
        sl_loc = jax.lax.dot_general(qh, k[:, sl], (((1,), (1,)), ((), ())),
                                     preferred_element_type=jnp.float32)
        e = jnp.where(causal, jnp.exp(sl_loc), 0.0)
        o_loc = jnp.dot(e, v[:, sl], preferred_element_type=jnp.float32)
        o_loc = o_loc * (1.0 / jnp.sum(e, axis=1, keepdims=True))
        sx = jax.lax.dot_general(qh, kx[:, sl], (((1,), (1,)), ((), ())),
                                 preferred_element_type=jnp.float32)
        ex = jnp.exp(sx)
        o_ext = jnp.dot(ex, vx[:, sl], preferred_element_type=jnp.float32)
        o_ext = o_ext * (1.0 / jnp.sum(ex, axis=1, keepdims=True))
        g = jax.nn.sigmoid(gate_ref[h])
        outs.append(o_loc * g + o_ext * (1.0 - g))
    o_ref[0] = jnp.concatenate(outs, axis=1)


def _attention(gate_vec, qf3, ckv, kx3, vx3):
    b, s, _ = qf3.shape
    sk = kx3.shape[1]
    return pl.pallas_call(
        _attn_body,
        grid=(b,),
        in_specs=[pl.BlockSpec(memory_space=pltpu.SMEM),
                  pl.BlockSpec((1, s, D), lambda i: (i, 0, 0)),
                  pl.BlockSpec((1, s, 2, D), lambda i: (i, 0, 0, 0)),
                  pl.BlockSpec((1, sk, D), lambda i: (i, 0, 0)),
                  pl.BlockSpec((1, sk, D), lambda i: (i, 0, 0))],
        out_specs=pl.BlockSpec((1, s, D), lambda i: (i, 0, 0)),
        out_shape=jax.ShapeDtypeStruct((b, s, D), jnp.float32),
    )(gate_vec, qf3, ckv, kx3, vx3)


def _outproj_body(a_ref, w_ref, b_ref, o_ref):
    o_ref[...] = jnp.dot(a_ref[...], w_ref[...],
                         preferred_element_type=jnp.float32) + b_ref[...]


def _outproj(a, WoT, bo):
    n = a.shape[0]
    blk = 256
    return pl.pallas_call(
        _outproj_body,
        grid=(n // blk,),
        in_specs=[pl.BlockSpec((blk, D), lambda i: (i, 0)),
                  pl.BlockSpec((D, EMB), lambda i: (0, 0)),
                  pl.BlockSpec((1, EMB), lambda i: (0, 0))],
        out_specs=pl.BlockSpec((blk, EMB), lambda i: (i, 0)),
        out_shape=jax.ShapeDtypeStruct((n, EMB), jnp.float32),
    )(a, WoT, bo)


def kernel(x, Wq, bq, Wk, bk, Wv, bv, Wo, bo, gate, kv_mem):
    b, s, _ = x.shape
    n = b * s
    xf = x.reshape(n, EMB)

    qf, current_kv = _projections(xf, Wq.T, Wk.T, Wv.T,
                                  bq.reshape(1, D), bk.reshape(1, D),
                                  bv.reshape(1, D), b, s)
    keys = kv_mem[:, 0, :]
    scores2, cm2 = _knn_scores(qf, keys)
    kx, vx = _sc_topk_gather(scores2, cm2, kv_mem)

    qkv = _attention(gate.reshape(H),
                     qf.reshape(b, s, D), current_kv,
                     kx.reshape(b, s * TOPK, D), vx.reshape(b, s * TOPK, D))

    out = _outproj(qkv.reshape(n, D), Wo.T, bo.reshape(1, EMB)).reshape(b, s, EMB)
    return out, current_kv


# speedup vs baseline: 1.0339x; 1.0339x over previous
import jax
import jax.numpy as jnp
from jax import lax
from jax.experimental import pallas as pl
from jax.experimental.pallas import tpu as pltpu
from jax.experimental.pallas import tpu_sc as plsc

H = 8
DH = 64
TOPK = 8
EMB = 512
M = 8192
D = H * DH
NEG = -3.0e38

CHUNK = 128
NCHUNK = M // CHUNK
NWORK = 32
L = 16


def _proj_body(x_ref, wq_ref, wk_ref, wv_ref, bq_ref, bk_ref, bv_ref,
               qf_ref, ckv_ref):
    xb = x_ref[...]
    q = jnp.dot(xb, wq_ref[...], preferred_element_type=jnp.float32) + bq_ref[...]
    k = jnp.dot(xb, wk_ref[...], preferred_element_type=jnp.float32) + bk_ref[...]
    v = jnp.dot(xb, wv_ref[...], preferred_element_type=jnp.float32) + bv_ref[...]
    qn = jnp.sqrt(jnp.sum(q * q, axis=1, keepdims=True))
    kn = jnp.sqrt(jnp.sum(k * k, axis=1, keepdims=True))
    qf_ref[...] = q / jnp.maximum(qn, 1e-12)
    ckv_ref[0, :, 0, :] = k / jnp.maximum(kn, 1e-12)
    ckv_ref[0, :, 1, :] = v


def _projections(xf, WqT, WkT, WvT, bq, bk, bv, b, s):
    n = xf.shape[0]
    blk = 256
    per_b = s // blk
    row_spec = pl.BlockSpec((blk, EMB), lambda i: (i, 0))
    w_spec = pl.BlockSpec((EMB, D), lambda i: (0, 0))
    b_spec = pl.BlockSpec((1, D), lambda i: (0, 0))
    return pl.pallas_call(
        _proj_body,
        grid=(n // blk,),
        in_specs=[row_spec, w_spec, w_spec, w_spec, b_spec, b_spec, b_spec],
        out_specs=[pl.BlockSpec((blk, D), lambda i: (i, 0)),
                   pl.BlockSpec((1, blk, 2, D),
                                lambda i: (i // per_b, i % per_b, 0, 0))],
        out_shape=[jax.ShapeDtypeStruct((n, D), jnp.float32),
                   jax.ShapeDtypeStruct((b, s, 2, D), jnp.float32)],
    )(xf, WqT, WkT, WvT, bq, bk, bv)


def _scores_body(qf_ref, keys_ref, s_ref, cm_ref):
    s = jax.lax.dot_general(qf_ref[...], keys_ref[...],
                            (((1,), (1,)), ((), ())),
                            preferred_element_type=jnp.float32)
    blk = s.shape[0]
    s_ref[...] = s.reshape(blk * NCHUNK, CHUNK)
    cm_ref[...] = jnp.max(s.reshape(blk, NCHUNK, CHUNK), axis=2).reshape(
        blk * NCHUNK // CHUNK, CHUNK)


def _knn_scores(qf, keys):
    n = qf.shape[0]
    blk = 64
    return pl.pallas_call(
        _scores_body,
        grid=(n // blk,),
        in_specs=[pl.BlockSpec((blk, D), lambda i: (i, 0)),
                  pl.BlockSpec((M, D), lambda i: (0, 0))],
        out_specs=[pl.BlockSpec((blk * NCHUNK, CHUNK), lambda i: (i, 0)),
                   pl.BlockSpec((blk * NCHUNK // CHUNK, CHUNK),
                                lambda i: (i, 0))],
        out_shape=[jax.ShapeDtypeStruct((n * NCHUNK, CHUNK), jnp.float32),
                   jax.ShapeDtypeStruct((n * NCHUNK // CHUNK, CHUNK),
                                        jnp.float32)],
    )(qf, keys)


def _sc_topk_gather_body(scores2_hbm, cm2_hbm, kv_hbm, kx_hbm, vx_hbm,
                         cmb, cbuf, cidx, cms, kvidx, gA, gB,
                         sgA, sgB, soA, soB):
    rpw = 2048 // NWORK
    npair = rpw // 2
    wid = lax.axis_index("s") * 2 + lax.axis_index("c")
    base = wid * rpw
    iota = lax.iota(jnp.int32, L)
    big = jnp.int32(1 << 30)
    cvec = NCHUNK // L

    pltpu.sync_copy(cm2_hbm.at[pl.ds(wid * npair, npair)], cmb)

    def sel_pair(p, carry):
        accP = jnp.zeros((L,), jnp.int32)
        cmsP = jnp.full((L,), NEG, jnp.float32)
        for half in range(2):
            r = 2 * p + half
            lb = TOPK * half
            cw = [cmb[p, pl.ds(half * NCHUNK + L * j, L)] for j in range(cvec)]
            for t in range(TOPK):
                m = jnp.max(jnp.maximum(jnp.maximum(cw[0], cw[1]),
                                        jnp.maximum(cw[2], cw[3])))
                cand = jnp.where(cw[0] == m, iota, big)
                for j in range(1, cvec):
                    cand = jnp.minimum(cand, jnp.where(cw[j] == m, iota + L * j, big))
                c = jnp.min(cand)
                accP = jnp.where(iota == lb + t, (base + r) * NCHUNK + c, accP)
                cmsP = jnp.where(iota == lb + t, m, cmsP)
                for j in range(cvec):
                    cw[j] = jnp.where(iota == c - L * j, NEG, cw[j])
        cidx[pl.ds(p * L, L)] = accP
        cms[pl.ds(p * L, L)] = cmsP
        return carry

    lax.fori_loop(0, npair, sel_pair, 0)

    for hf in range(2):
        po = hf * (npair // 2)
        coff = po * L
        nf = npair // 2 * L
        for qd in range(nf // CHUNK):
            pltpu.async_copy(
                scores2_hbm.at[cidx.at[pl.ds(coff + qd * CHUNK, CHUNK)]],
                cbuf.at[pl.ds(qd * CHUNK, CHUNK)], sgA)
        for qd in range(nf // CHUNK):
            pltpu.make_async_copy(
                scores2_hbm.at[cidx.at[pl.ds(coff + qd * CHUNK, CHUNK)]],
                cbuf.at[pl.ds(qd * CHUNK, CHUNK)], sgA).wait()

        def top_pair(p, carry, po=po):
            cmsv = cms[pl.ds(p * L, L)]
            fidv = cidx[pl.ds(p * L, L)]
            accK = jnp.zeros((L,), jnp.int32)
            for half in range(2):
                r = 2 * p + half
                lb = TOPK * half
                inr = (iota >= lb) & (iota < lb + TOPK)
                cm8 = jnp.where(inr, cmsv, NEG)
                for t in range(TOPK):
                    m = jnp.max(cm8)
                    c = jnp.min(jnp.where(cm8 == m, iota, big))
                    slot = (p - po) * L + c
                    fid = jnp.min(jnp.where(iota == c, fidv, big))
                    c_loc = fid - (base + r) * NCHUNK
                    vs = [cbuf[slot, pl.ds(L * j, L)] for j in range(CHUNK // L)]
                    ecand = jnp.where(vs[0] == m, iota, big)
                    for j in range(1, CHUNK // L):
                        ecand = jnp.minimum(ecand,
                                            jnp.where(vs[j] == m, iota + L * j, big))
                    e = jnp.min(ecand)
                    accK = jnp.where(iota == lb + t, c_loc * CHUNK + e, accK)
                    nmv = None
                    for j in range(CHUNK // L):
                        vj = jnp.where(iota == e - L * j, NEG, vs[j])
                        cbuf[slot, pl.ds(L * j, L)] = vj
                        nmv = vj if nmv is None else jnp.maximum(nmv, vj)
                    cm8 = jnp.where(iota == c, jnp.max(nmv), cm8)
            kvidx[pl.ds(p * L, L)] = accK
            return carry

        lax.fori_loop(po, po + npair // 2, top_pair, 0)

    def issue_gather(p, buf, sem):
        pltpu.async_copy(kv_hbm.at[kvidx.at[pl.ds(p * L, L)]], buf, sem)

    def drain_gather(p, buf, sem):
        pltpu.make_async_copy(kv_hbm.at[kvidx.at[pl.ds(p * L, L)]], buf, sem).wait()

    def issue_out(p, buf, sem):
        dst = pl.ds((base + 2 * p) * TOPK, 2 * TOPK)
        pltpu.async_copy(buf.at[:, 0, :], kx_hbm.at[dst], sem)
        pltpu.async_copy(buf.at[:, 1, :], vx_hbm.at[dst], sem)

    def drain_out(p, buf, sem):
        dst = pl.ds((base + 2 * p) * TOPK, 2 * TOPK)
        pltpu.make_async_copy(buf.at[:, 0, :], kx_hbm.at[dst], sem).wait()
        pltpu.make_async_copy(buf.at[:, 1, :], vx_hbm.at[dst], sem).wait()

    issue_gather(0, gA, sgA)

    def g_body(i, carry):
        pA = 2 * i
        pB = 2 * i + 1
        issue_gather(pB, gB, sgB)
        drain_gather(pA, gA, sgA)
        issue_out(pA, gA, soA)
        drain_gather(pB, gB, sgB)
        issue_out(pB, gB, soB)
        drain_out(pA, gA, soA)

        @pl.when(i < npair // 2 - 1)
        def _():
            issue_gather(pA + 2, gA, sgA)
        drain_out(pB, gB, soB)
        return carry

    lax.fori_loop(0, npair // 2, g_body, 0)


def _sc_topk_gather(scores2, cm2, kv_mem):
    n = scores2.shape[0] // NCHUNK
    rpw = n // NWORK
    mesh = plsc.VectorSubcoreMesh(core_axis_name="c", subcore_axis_name="s")
    fn = pl.kernel(
        _sc_topk_gather_body,
        out_type=[jax.ShapeDtypeStruct((n * TOPK, D), jnp.float32),
                  jax.ShapeDtypeStruct((n * TOPK, D), jnp.float32)],
        mesh=mesh,
        scratch_types=[
            pltpu.VMEM((rpw // 2, CHUNK), jnp.float32),
            pltpu.VMEM((rpw * TOPK // 2, CHUNK), jnp.float32),
            pltpu.VMEM((rpw * TOPK,), jnp.int32),
            pltpu.VMEM((rpw * TOPK,), jnp.float32),
            pltpu.VMEM((rpw * TOPK,), jnp.int32),
            pltpu.VMEM((2 * TOPK, 2, D), jnp.float32),
            pltpu.VMEM((2 * TOPK, 2, D), jnp.float32),
            pltpu.SemaphoreType.DMA,
            pltpu.SemaphoreType.DMA,
            pltpu.SemaphoreType.DMA,
            pltpu.SemaphoreType.DMA,
        ],
        compiler_params=pltpu.CompilerParams(needs_layout_passes=False),
    )
    return fn(scores2, cm2, kv_mem)


def _attn_local_body(gate_ref, q_ref, ckv_ref, o_ref):
    scale = DH ** -0.5
    q = q_ref[0] * scale
    k = ckv_ref[0, :, 0, :]
    v = ckv_ref[0, :, 1, :]
    s = q.shape[0]
    ri = jax.lax.broadcasted_iota(jnp.int32, (s, s), 0)
    ci = jax.lax.broadcasted_iota(jnp.int32, (s, s), 1)
    causal = ci <= ri
    outs = []
    for h in range(H):
        sl = slice(h * DH, (h + 1) * DH)
        qh = q[:, sl]
        sl_loc = jax.lax.dot_general(qh, k[:, sl], (((1,), (1,)), ((), ())),
                                     preferred_element_type=jnp.float32)
        e = jnp.where(causal, jnp.exp(sl_loc), 0.0)
        o_loc = jnp.dot(e, v[:, sl], preferred_element_type=jnp.float32)
        g = jax.nn.sigmoid(gate_ref[h])
        outs.append(o_loc * (g / jnp.sum(e, axis=1, keepdims=True)))
    o_ref[0] = jnp.concatenate(outs, axis=1)


def _attn_local(gate_vec, qf3, ckv):
    b, s, _ = qf3.shape
    return pl.pallas_call(
        _attn_local_body,
        grid=(b,),
        in_specs=[pl.BlockSpec(memory_space=pltpu.SMEM),
                  pl.BlockSpec((1, s, D), lambda i: (i, 0, 0)),
                  pl.BlockSpec((1, s, 2, D), lambda i: (i, 0, 0, 0))],
        out_specs=pl.BlockSpec((1, s, D), lambda i: (i, 0, 0)),
        out_shape=jax.ShapeDtypeStruct((b, s, D), jnp.float32),
    )(gate_vec, qf3, ckv)


def _attn_ext_body(gate_ref, q_ref, kx_ref, vx_ref, oloc_ref, o_ref):
    scale = DH ** -0.5
    q = q_ref[0] * scale
    kx = kx_ref[0]
    vx = vx_ref[0]
    outs = []
    for h in range(H):
        sl = slice(h * DH, (h + 1) * DH)
        qh = q[:, sl]
        sx = jax.lax.dot_general(qh, kx[:, sl], (((1,), (1,)), ((), ())),
                                 preferred_element_type=jnp.float32)
        ex = jnp.exp(sx)
        o_ext = jnp.dot(ex, vx[:, sl], preferred_element_type=jnp.float32)
        g = jax.nn.sigmoid(gate_ref[h])
        outs.append(o_ext * ((1.0 - g) / jnp.sum(ex, axis=1, keepdims=True)))
    o_ref[0] = oloc_ref[0] + jnp.concatenate(outs, axis=1)


def _attn_ext(gate_vec, qf3, kx3, vx3, oloc):
    b, s, _ = qf3.shape
    sk = kx3.shape[1]
    return pl.pallas_call(
        _attn_ext_body,
        grid=(b,),
        in_specs=[pl.BlockSpec(memory_space=pltpu.SMEM),
                  pl.BlockSpec((1, s, D), lambda i: (i, 0, 0)),
                  pl.BlockSpec((1, sk, D), lambda i: (i, 0, 0)),
                  pl.BlockSpec((1, sk, D), lambda i: (i, 0, 0)),
                  pl.BlockSpec((1, s, D), lambda i: (i, 0, 0))],
        out_specs=pl.BlockSpec((1, s, D), lambda i: (i, 0, 0)),
        out_shape=jax.ShapeDtypeStruct((b, s, D), jnp.float32),
    )(gate_vec, qf3, kx3, vx3, oloc)


def _outproj_body(a_ref, w_ref, b_ref, o_ref):
    o_ref[...] = jnp.dot(a_ref[...], w_ref[...],
                         preferred_element_type=jnp.float32) + b_ref[...]


def _outproj(a, WoT, bo):
    n = a.shape[0]
    blk = 256
    return pl.pallas_call(
        _outproj_body,
        grid=(n // blk,),
        in_specs=[pl.BlockSpec((blk, D), lambda i: (i, 0)),
                  pl.BlockSpec((D, EMB), lambda i: (0, 0)),
                  pl.BlockSpec((1, EMB), lambda i: (0, 0))],
        out_specs=pl.BlockSpec((blk, EMB), lambda i: (i, 0)),
        out_shape=jax.ShapeDtypeStruct((n, EMB), jnp.float32),
    )(a, WoT, bo)


def kernel(x, Wq, bq, Wk, bk, Wv, bv, Wo, bo, gate, kv_mem):
    b, s, _ = x.shape
    n = b * s
    xf = x.reshape(n, EMB)

    qf, current_kv = _projections(xf, Wq.T, Wk.T, Wv.T,
                                  bq.reshape(1, D), bk.reshape(1, D),
                                  bv.reshape(1, D), b, s)
    keys = kv_mem[:, 0, :]
    scores2, cm2 = _knn_scores(qf, keys)
    kx, vx = _sc_topk_gather(scores2, cm2, kv_mem)

    gate_vec = gate.reshape(H)
    qf3 = qf.reshape(b, s, D)
    oloc = _attn_local(gate_vec, qf3, current_kv)
    qkv = _attn_ext(gate_vec, qf3, kx.reshape(b, s * TOPK, D),
                    vx.reshape(b, s * TOPK, D), oloc)

    out = _outproj(qkv.reshape(n, D), Wo.T, bo.reshape(1, EMB)).reshape(b, s, EMB)
    return out, current_kv
